# baseline (speedup 1.0000x reference)
import functools

import jax
import jax.numpy as jnp
from jax import lax
from jax.experimental import pallas as pl
from jax.experimental.pallas import tpu as pltpu
from jax.experimental.pallas import tpu_sc as plsc

N_ITEMS = 1000000
N_FACTORS = 64
BATCH = 16384

_info = plsc.get_sparse_core_info()
NC, NS, L = _info.num_cores, _info.num_subcores, _info.num_lanes
NW = NC * NS

NJ = 7813
CPT = 245
NCOLS = 5
W = NCOLS * 128
NCHUNK = (CPT + NCOLS - 1) // NCOLS
JC_MAX = NJ - NCOLS

PAIR_CAP = 1536
ACT_CAP = 128
NPR = BATCH * 2 + 8
DUMP = BATCH * 2

ICH = 2048


def _extract_body(items0_hbm, items1_hbm, table_hbm, rows_hbm,
                  sbuf, ibuf, pairs_r, pairs_pay, actv_r, actv_pay,
                  stag, payidx, sem_in, sem_sc, sem_i):
    cid = lax.axis_index("c")
    sid = lax.axis_index("s")
    wid = sid * NC + cid
    jlo = wid * CPT
    jhi = jlo + CPT

    lane = lax.iota(jnp.int32, L)

    def compact_chunk(np_cur, items_hbm, slot, cb):
        pltpu.async_copy(items_hbm.at[pl.ds(cb, ICH)], ibuf, sem_i).wait()

        def vbody(v, np_c):
            r16 = ibuf[pl.ds(pl.multiple_of(v * L, L), L)]
            j16 = lax.shift_right_logical(r16, 7)
            mask = (j16 >= jlo) & (j16 < jhi)
            pay16 = jnp.full((L,), slot * BATCH + cb, jnp.int32) + v * L + lane
            off = pl.multiple_of(0, 1) + np_c
            plsc.store_compressed(pairs_r.at[pl.ds(off, L)], r16, mask=mask)
            plsc.store_compressed(pairs_pay.at[pl.ds(off, L)], pay16, mask=mask)
            cnt = plsc.all_reduce_population_count(mask)[0]
            return np_c + cnt

        return lax.fori_loop(0, ICH // L, vbody, np_cur, unroll=False)

    np_total = 0
    for slot, items_hbm in ((0, items0_hbm), (1, items1_hbm)):
        for c in range(BATCH // ICH):
            np_total = compact_chunk(np_total, items_hbm, slot, c * ICH)
    np_total = jnp.minimum(np_total, PAIR_CAP - L)

    kvecs = [lane + g * L for g in range(4)]
    i_vecs = [lax.shift_right_logical(k, 3) for k in kvecs]
    d_vecs = [lax.bitwise_and(k, 7) for k in kvecs]

    def fire_chunk(ch):
        jc_eff = jnp.minimum(jlo + ch * NCOLS, JC_MAX)
        buf = lax.rem(ch, 2)
        for i in range(8):
            pltpu.async_copy(
                table_hbm.at[pl.ds(i * 8, 8),
                             pl.ds(jc_eff * 128, W)],
                sbuf.at[buf, i], sem_in)
        return jc_eff

    fire_chunk(jnp.int32(0))

    def chunk_body(ch, _):
        jc = jlo + ch * NCOLS
        jc_eff = jnp.minimum(jc, JC_MAX)
        buf = lax.rem(ch, 2)
        for i in range(8):
            pltpu.make_async_copy(
                table_hbm.at[pl.ds(i * 8, 8), pl.ds(0, W)],
                sbuf.at[buf, i], sem_in).wait()

        @pl.when(ch + 1 < NCHUNK)
        def _():
            fire_chunk(ch + 1)

        def sel_body(q, na_c):
            off_q = pl.multiple_of(0, 1) + q * L
            r16 = pairs_r[pl.ds(off_q, L)]
            p16 = pairs_pay[pl.ds(off_q, L)]
            j16 = lax.shift_right_logical(r16, 7)
            valid = (q * L + lane) < np_total
            mask = valid & (j16 >= jc) & (j16 < jc + NCOLS)
            off = pl.multiple_of(0, 1) + na_c
            plsc.store_compressed(actv_r.at[pl.ds(off, L)], r16, mask=mask)
            plsc.store_compressed(actv_pay.at[pl.ds(off, L)], p16, mask=mask)
            cnt = plsc.all_reduce_population_count(mask)[0]
            return na_c + cnt

        fill_r = jnp.full((L,), 0, jnp.int32) + jc_eff * 128
        fill_p = jnp.full((L,), DUMP, jnp.int32)
        for q in range(ACT_CAP // L):
            actv_r[pl.ds(q * L, L)] = fill_r
            actv_pay[pl.ds(q * L, L)] = fill_p

        nq = lax.shift_right_logical(np_total + L - 1, 4)
        na = lax.fori_loop(0, nq, sel_body, 0, unroll=False)
        na = jnp.minimum(na, ACT_CAP - L)
        ng = lax.shift_right_logical(na + L - 1, 4)

        def grp_body(g, _g):
            goff = pl.multiple_of(0, 1) + g * L
            r16 = actv_r[pl.ds(goff, L)]
            p16 = actv_pay[pl.ds(goff, L)]
            payidx[...] = p16
            for l in range(L):
                pos = r16[l] - jc_eff * 128
                posv = jnp.full((L,), 0, jnp.int32) + pos
                for gg in range(4):
                    vals = plsc.load_gather(
                        sbuf.at[buf], [i_vecs[gg], d_vecs[gg], posv])
                    stag[l, pl.ds(gg * L, L)] = vals
            cp = pltpu.async_copy(
                stag, rows_hbm.at[payidx], sem_sc)
            cp.wait()
            return 0

        lax.fori_loop(0, ng, grp_body, 0, unroll=False)
        return 0

    lax.fori_loop(0, NCHUNK, chunk_body, 0, unroll=False)


@jax.jit
def _sc_extract(items0, items1, table_t):
    mesh = plsc.VectorSubcoreMesh(core_axis_name="c", subcore_axis_name="s")
    f = functools.partial(
        pl.kernel,
        out_type=jax.ShapeDtypeStruct((NPR, 128), jnp.float32),
        mesh=mesh,
        scratch_types=[
            pltpu.VMEM((2, 8, 8, W), jnp.float32),
            pltpu.VMEM((ICH,), jnp.int32),
            pltpu.VMEM((PAIR_CAP,), jnp.int32),
            pltpu.VMEM((PAIR_CAP,), jnp.int32),
            pltpu.VMEM((ACT_CAP,), jnp.int32),
            pltpu.VMEM((ACT_CAP,), jnp.int32),
            pltpu.VMEM((L, 128), jnp.float32),
            pltpu.VMEM((L,), jnp.int32),
            pltpu.SemaphoreType.DMA,
            pltpu.SemaphoreType.DMA,
            pltpu.SemaphoreType.DMA,
        ],
        compiler_params=pltpu.CompilerParams(
            use_tc_tiling_on_sc=True,
            needs_layout_passes=False,
        ),
    )(_extract_body)
    return f(items0, items1, table_t)


def _combine_body(a_ref, b_ref, o_ref):
    x = a_ref[:, :N_FACTORS] * b_ref[:, :N_FACTORS]
    s = jnp.sum(x, axis=1)
    o_ref[...] = 1.0 / (1.0 + jnp.exp(-s))


_BLK = 2048


@jax.jit
def _tc_combine(rows):
    return pl.pallas_call(
        _combine_body,
        grid=(BATCH // _BLK,),
        in_specs=[
            pl.BlockSpec((_BLK, 128), lambda i: (i, 0)),
            pl.BlockSpec((_BLK, 128), lambda i: (i + BATCH // _BLK, 0)),
        ],
        out_specs=pl.BlockSpec((_BLK,), lambda i: (i,)),
        out_shape=jax.ShapeDtypeStruct((BATCH,), jnp.float32),
    )(rows, rows)


def kernel(items, item_factors):
    items0 = items[0].astype(jnp.int32)
    items1 = items[1].astype(jnp.int32)
    table_t = item_factors.T
    rows = _sc_extract(items0, items1, table_t)
    return _tc_combine(rows)


# speedup vs baseline: 1.0137x; 1.0137x over previous
import functools

import jax
import jax.numpy as jnp
from jax import lax
from jax.experimental import pallas as pl
from jax.experimental.pallas import tpu as pltpu
from jax.experimental.pallas import tpu_sc as plsc

N_ITEMS = 1000000
N_FACTORS = 64
BATCH = 16384

_info = plsc.get_sparse_core_info()
NC, NS, L = _info.num_cores, _info.num_subcores, _info.num_lanes
NW = NC * NS

NJ = 7813
CPT = 245
NCOLS = 5
W = NCOLS * 128
NCHUNK = (CPT + NCOLS - 1) // NCOLS
JC_MAX = NJ - NCOLS

PAIR_CAP = 1536
ACT_CAP = 128
NPR = BATCH * 2 + 8
DUMP = BATCH * 2

ICH = 2048


def _extract_body(items0_hbm, items1_hbm, table_hbm, rows_hbm,
                  sbuf, ibuf, pairs_r, pairs_pay, actv_r, actv_pay,
                  stag, payidx, sem_in, sem_sc, sem_i):
    cid = lax.axis_index("c")
    sid = lax.axis_index("s")
    wid = sid * NC + cid
    jlo = wid * CPT
    jhi = jlo + CPT

    lane = lax.iota(jnp.int32, L)

    def compact_chunk(np_cur, items_hbm, slot, cb, ib, nxt):
        pltpu.make_async_copy(
            items_hbm.at[pl.ds(0, ICH)], ibuf.at[ib], sem_i).wait()
        if nxt is not None:
            pltpu.async_copy(
                nxt[0].at[pl.ds(nxt[1], ICH)], ibuf.at[1 - ib], sem_i)

        def vbody(v, np_c):
            r16 = ibuf[ib, pl.ds(pl.multiple_of(v * L, L), L)]
            j16 = lax.shift_right_logical(r16, 7)
            mask = (j16 >= jlo) & (j16 < jhi)
            pay16 = jnp.full((L,), slot * BATCH + cb, jnp.int32) + v * L + lane
            off = pl.multiple_of(0, 1) + np_c
            plsc.store_compressed(pairs_r.at[pl.ds(off, L)], r16, mask=mask)
            plsc.store_compressed(pairs_pay.at[pl.ds(off, L)], pay16, mask=mask)
            cnt = plsc.all_reduce_population_count(mask)[0]
            return np_c + cnt

        return lax.fori_loop(0, ICH // L, vbody, np_cur, unroll=False)

    np_total = 0
    steps = [(slot, ih, c * ICH)
             for slot, ih in ((0, items0_hbm), (1, items1_hbm))
             for c in range(BATCH // ICH)]
    pltpu.async_copy(items0_hbm.at[pl.ds(0, ICH)], ibuf.at[0], sem_i)
    for k, (slot, ih, cb) in enumerate(steps):
        nxt = (steps[k + 1][1], steps[k + 1][2]) if k + 1 < len(steps) else None
        np_total = compact_chunk(np_total, ih, slot, cb, k % 2, nxt)
    np_total = jnp.minimum(np_total, PAIR_CAP - L)

    kvecs = [lane + g * L for g in range(4)]
    i_vecs = [lax.shift_right_logical(k, 3) for k in kvecs]
    d_vecs = [lax.bitwise_and(k, 7) for k in kvecs]

    def fire_chunk(ch):
        jc_eff = jnp.minimum(jlo + ch * NCOLS, JC_MAX)
        buf = lax.rem(ch, 2)
        for i in range(8):
            pltpu.async_copy(
                table_hbm.at[pl.ds(i * 8, 8),
                             pl.ds(jc_eff * 128, W)],
                sbuf.at[buf, i], sem_in)
        return jc_eff

    fire_chunk(jnp.int32(0))

    def chunk_body(ch, _):
        jc = jlo + ch * NCOLS
        jc_eff = jnp.minimum(jc, JC_MAX)
        buf = lax.rem(ch, 2)
        for i in range(8):
            pltpu.make_async_copy(
                table_hbm.at[pl.ds(i * 8, 8), pl.ds(0, W)],
                sbuf.at[buf, i], sem_in).wait()

        @pl.when(ch + 1 < NCHUNK)
        def _():
            fire_chunk(ch + 1)

        def sel_body(q, na_c):
            off_q = pl.multiple_of(0, 1) + q * L
            r16 = pairs_r[pl.ds(off_q, L)]
            p16 = pairs_pay[pl.ds(off_q, L)]
            j16 = lax.shift_right_logical(r16, 7)
            valid = (q * L + lane) < np_total
            mask = valid & (j16 >= jc) & (j16 < jc + NCOLS)
            off = pl.multiple_of(0, 1) + na_c
            plsc.store_compressed(actv_r.at[pl.ds(off, L)], r16, mask=mask)
            plsc.store_compressed(actv_pay.at[pl.ds(off, L)], p16, mask=mask)
            cnt = plsc.all_reduce_population_count(mask)[0]
            return na_c + cnt

        fill_r = jnp.full((L,), 0, jnp.int32) + jc_eff * 128
        fill_p = jnp.full((L,), DUMP, jnp.int32)
        for q in range(ACT_CAP // L):
            actv_r[pl.ds(q * L, L)] = fill_r
            actv_pay[pl.ds(q * L, L)] = fill_p

        nq = lax.shift_right_logical(np_total + L - 1, 4)
        na = lax.fori_loop(0, nq, sel_body, 0, unroll=False)
        na = jnp.minimum(na, ACT_CAP - L)
        ng = lax.shift_right_logical(na + L - 1, 4)

        def drain_one():
            pltpu.make_async_copy(
                rows_hbm.at[pl.ds(0, L)], stag.at[0], sem_sc).wait()

        def grp_body(g, _g):
            slot_g = lax.rem(g, 4)

            @pl.when(g >= 4)
            def _():
                drain_one()

            goff = pl.multiple_of(0, 1) + g * L
            r16 = actv_r[pl.ds(goff, L)]
            p16 = actv_pay[pl.ds(goff, L)]
            payidx[slot_g] = p16
            for l in range(L):
                pos = r16[l] - jc_eff * 128
                posv = jnp.full((L,), 0, jnp.int32) + pos
                for gg in range(4):
                    vals = plsc.load_gather(
                        sbuf.at[buf], [i_vecs[gg], d_vecs[gg], posv])
                    stag[slot_g, l, pl.ds(gg * L, L)] = vals
            pltpu.async_copy(
                stag.at[slot_g], rows_hbm.at[payidx.at[slot_g]], sem_sc)
            return 0

        lax.fori_loop(0, ng, grp_body, 0, unroll=False)

        def drain_body(g, _g):
            drain_one()
            return 0

        lax.fori_loop(0, jnp.minimum(ng, 4), drain_body, 0, unroll=False)
        return 0

    lax.fori_loop(0, NCHUNK, chunk_body, 0, unroll=False)


@jax.jit
def _sc_extract(items0, items1, table_t):
    mesh = plsc.VectorSubcoreMesh(core_axis_name="c", subcore_axis_name="s")
    f = functools.partial(
        pl.kernel,
        out_type=jax.ShapeDtypeStruct((NPR, 128), jnp.float32),
        mesh=mesh,
        scratch_types=[
            pltpu.VMEM((2, 8, 8, W), jnp.float32),
            pltpu.VMEM((2, ICH), jnp.int32),
            pltpu.VMEM((PAIR_CAP,), jnp.int32),
            pltpu.VMEM((PAIR_CAP,), jnp.int32),
            pltpu.VMEM((ACT_CAP,), jnp.int32),
            pltpu.VMEM((ACT_CAP,), jnp.int32),
            pltpu.VMEM((4, L, 128), jnp.float32),
            pltpu.VMEM((4, L), jnp.int32),
            pltpu.SemaphoreType.DMA,
            pltpu.SemaphoreType.DMA,
            pltpu.SemaphoreType.DMA,
        ],
        compiler_params=pltpu.CompilerParams(
            use_tc_tiling_on_sc=True,
            needs_layout_passes=False,
        ),
    )(_extract_body)
    return f(items0, items1, table_t)


def _combine_body(a_ref, b_ref, o_ref):
    x = a_ref[:, :N_FACTORS] * b_ref[:, :N_FACTORS]
    s = jnp.sum(x, axis=1)
    o_ref[...] = 1.0 / (1.0 + jnp.exp(-s))


_BLK = 2048


@jax.jit
def _tc_combine(rows):
    return pl.pallas_call(
        _combine_body,
        grid=(BATCH // _BLK,),
        in_specs=[
            pl.BlockSpec((_BLK, 128), lambda i: (i, 0)),
            pl.BlockSpec((_BLK, 128), lambda i: (i + BATCH // _BLK, 0)),
        ],
        out_specs=pl.BlockSpec((_BLK,), lambda i: (i,)),
        out_shape=jax.ShapeDtypeStruct((BATCH,), jnp.float32),
    )(rows, rows)


def kernel(items, item_factors):
    items0 = items[0].astype(jnp.int32)
    items1 = items[1].astype(jnp.int32)
    table_t = item_factors.T
    rows = _sc_extract(items0, items1, table_t)
    return _tc_combine(rows)
